# baseline (speedup 1.0000x reference)
import jax
import jax.numpy as jnp
from jax import lax
from jax.experimental import pallas as pl
from jax.experimental.pallas import tpu as pltpu
from jax.experimental.pallas import tpu_sc as plsc

TOTAL = 8192
D = 1024
NB = 8
NC = 2
NS = 16
NW = NC * NS
TC_ROWS = 5120
TCR = 1024
SC_ROWS = TOTAL - TC_ROWS
RPW = SC_ROWS // NW
CH = RPW // 2
NCHUNK = RPW // CH
L = 16
G = 2
GW = D // G
GS = GW // L
STRIP = NB * D // NS


def _lane_extract(vec, lane, i):
    return jnp.sum(jnp.where(lane == i, vec, 0))


def _sc_body(mes_hbm, yv_hbm, cu_hbm,
             pacc_out, yvcas_out,
             buf0, buf1, acc, cuv, yvbuf,
             sem0, sem1, semyv):
    c = lax.axis_index("c")
    s = lax.axis_index("s")
    wid = s * NC + c
    base = TC_ROWS + wid * RPW

    bufs = (buf0, buf1)
    sems = (sem0, sem1)
    copies = [None, None]
    copies[0] = pltpu.async_copy(
        mes_hbm.at[pl.ds(base, CH)], buf0, sem0)
    if NCHUNK > 1:
        copies[1] = pltpu.async_copy(
            mes_hbm.at[pl.ds(base + CH, CH)], buf1, sem1)

    pltpu.sync_copy(cu_hbm, cuv.at[pl.ds(0, NB + 1)])

    yv_worker = jnp.logical_and(c == 0, s == 1)
    yv_copy = [None]

    @pl.when(yv_worker)
    def _():
        yv_copy[0] = pltpu.async_copy(
            yv_hbm.at[cuv.at[pl.ds(0, NB)]], yvbuf, semyv)

    def z_body(i, _):
        for k in range(8):
            acc[pl.ds(i * 128 + k * L, L)] = jnp.zeros((L,), jnp.float32)
        return 0
    lax.fori_loop(0, NB * D // 128, z_body, 0)

    cu_val = cuv[...]
    lane = lax.iota(jnp.int32, L)
    cub = [_lane_extract(cu_val, lane, b) for b in range(1, NB + 1)]

    def seg_of(pos):
        seg = jnp.int32(0)
        for b in range(NB - 1):
            seg = seg + (cub[b] <= pos).astype(jnp.int32)
        return seg

    for j in range(NCHUNK):
        p = j % 2
        copies[p].wait()
        buf = bufs[p]
        cstart = base + j * CH

        sfirst = seg_of(cstart)
        slast = seg_of(cstart + (CH - 1))

        def b_body(b, _, buf=buf, cstart=cstart):
            cu_lo = _lane_extract(cu_val, lane, b)
            cu_hi = _lane_extract(cu_val, lane, b + 1)
            lo = jnp.clip(cu_lo - cstart, 0, CH)
            hi = jnp.clip(cu_hi - cstart, 0, CH)
            for g in range(G):
                def r_body(r, carry, buf=buf, g=g):
                    return tuple(
                        carry[k] + buf[r, pl.ds(g * GW + k * L, L)]
                        for k in range(GS)
                    )
                carry = lax.fori_loop(
                    lo, hi, r_body,
                    tuple(jnp.zeros((L,), jnp.float32) for _ in range(GS)))
                for k in range(GS):
                    sl = pl.ds(b * D + g * GW + k * L, L)
                    acc[sl] = acc[sl] + carry[k]
            return 0

        lax.fori_loop(sfirst, slast + 1, b_body, 0)

    def w_body(b, _):
        pltpu.async_copy(acc.at[pl.ds(b * D, D)], pacc_out.at[wid, b], sem0)
        return 0
    lax.fori_loop(0, NB, w_body, 0)

    def wdrain_body(b, _):
        pltpu.make_async_copy(acc.at[pl.ds(0, D)], pacc_out.at[0, 0],
                              sem0).wait()
        return 0
    lax.fori_loop(0, NB, wdrain_body, 0)

    @pl.when(yv_worker)
    def _():
        yv_copy[0].wait()
        pltpu.sync_copy(yvbuf, yvcas_out)


def _tc_sum_body(cu_ref, mes_ref, out_ref):
    j = pl.program_id(0)

    @pl.when(j == 0)
    def _():
        out_ref[...] = jnp.zeros_like(out_ref)

    rows = lax.broadcasted_iota(jnp.int32, (NB, TCR), 1) + j * TCR
    lo = jnp.stack([cu_ref[b] for b in range(NB)])[:, None]
    hi = jnp.stack([cu_ref[b + 1] for b in range(NB)])[:, None]
    m = jnp.logical_and(lo <= rows, rows < hi).astype(jnp.float32)
    out_ref[...] += lax.dot(m, mes_ref[...],
                            preferred_element_type=jnp.float32)


def _tc_combine_body(cu_ref, pacc_ref, ptc_ref, out_ref):
    cnt = jnp.stack([cu_ref[b + 1] - cu_ref[b] for b in range(NB)])
    cntf = cnt[:, None].astype(jnp.float32)
    total = jnp.sum(pacc_ref[...], axis=0) + ptc_ref[...]
    out_ref[...] = total / cntf


@jax.jit
def _run(mes_update, yv, cu_seqlens):
    mesh = plsc.VectorSubcoreMesh(core_axis_name="c", subcore_axis_name="s", num_cores=NC)

    params = pltpu.CompilerParams(needs_layout_passes=False)
    sc_kernel = pl.kernel(
        _sc_body,
        mesh=mesh,
        compiler_params=params,
        out_type=[
            jax.ShapeDtypeStruct((NW, NB, D), jnp.float32),
            jax.ShapeDtypeStruct((NB, D), jnp.float32),
        ],
        scratch_types=[
            pltpu.VMEM((CH, D), jnp.float32),
            pltpu.VMEM((CH, D), jnp.float32),
            pltpu.VMEM((NB * D,), jnp.float32),
            pltpu.VMEM((L,), jnp.int32),
            pltpu.VMEM((NB, D), jnp.float32),
            pltpu.SemaphoreType.DMA,
            pltpu.SemaphoreType.DMA,
            pltpu.SemaphoreType.DMA,
        ],
    )
    ptc = pl.pallas_call(
        _tc_sum_body,
        grid=(TC_ROWS // TCR,),
        in_specs=[
            pl.BlockSpec(memory_space=pltpu.SMEM),
            pl.BlockSpec((TCR, D), lambda j: (j, 0)),
        ],
        out_specs=pl.BlockSpec((NB, D), lambda j: (0, 0)),
        out_shape=jax.ShapeDtypeStruct((NB, D), jnp.float32),
    )(cu_seqlens, mes_update)

    pacc, yv_cas = sc_kernel(mes_update, yv, cu_seqlens)

    cas_mean = pl.pallas_call(
        _tc_combine_body,
        in_specs=[
            pl.BlockSpec(memory_space=pltpu.SMEM),
            pl.BlockSpec((NW, NB, D)),
            pl.BlockSpec((NB, D)),
        ],
        out_specs=pl.BlockSpec((NB, D)),
        out_shape=jax.ShapeDtypeStruct((NB, D), jnp.float32),
    )(cu_seqlens, pacc, ptc)

    return cas_mean, yv_cas


def kernel(mes_update, yv, cu_seqlens):
    return _run(mes_update, yv, cu_seqlens.astype(jnp.int32))


# speedup vs baseline: 1.0720x; 1.0720x over previous
import jax
import jax.numpy as jnp
from jax import lax
from jax.experimental import pallas as pl
from jax.experimental.pallas import tpu as pltpu
from jax.experimental.pallas import tpu_sc as plsc

TOTAL = 8192
D = 1024
NB = 8
NC = 2
NS = 16
NW = NC * NS
TC_ROWS = 6144
TCR = 2048
SC_ROWS = TOTAL - TC_ROWS
RPW = SC_ROWS // NW
CH = RPW
NCHUNK = RPW // CH
L = 16
G = 2
GW = D // G
GS = GW // L
STRIP = NB * D // NS


def _lane_extract(vec, lane, i):
    return jnp.sum(jnp.where(lane == i, vec, 0))


def _sc_body(mes_hbm, yv_hbm, cu_hbm,
             pacc_out, yvcas_out,
             buf0, buf1, acc, cuv, yvbuf,
             sem0, sem1, semyv):
    c = lax.axis_index("c")
    s = lax.axis_index("s")
    wid = s * NC + c
    base = TC_ROWS + wid * RPW

    bufs = (buf0, buf1)
    sems = (sem0, sem1)
    copies = [None, None]
    copies[0] = pltpu.async_copy(
        mes_hbm.at[pl.ds(base, CH)], buf0, sem0)
    if NCHUNK > 1:
        copies[1] = pltpu.async_copy(
            mes_hbm.at[pl.ds(base + CH, CH)], buf1, sem1)

    pltpu.sync_copy(cu_hbm, cuv.at[pl.ds(0, NB + 1)])

    yv_worker = jnp.logical_and(c == 0, s == 1)
    yv_copy = [None]

    @pl.when(yv_worker)
    def _():
        yv_copy[0] = pltpu.async_copy(
            yv_hbm.at[cuv.at[pl.ds(0, NB)]], yvbuf, semyv)

    def z_body(i, _):
        for k in range(8):
            acc[pl.ds(i * 128 + k * L, L)] = jnp.zeros((L,), jnp.float32)
        return 0
    lax.fori_loop(0, NB * D // 128, z_body, 0)

    cu_val = cuv[...]
    lane = lax.iota(jnp.int32, L)
    cub = [_lane_extract(cu_val, lane, b) for b in range(1, NB + 1)]

    def seg_of(pos):
        seg = jnp.int32(0)
        for b in range(NB - 1):
            seg = seg + (cub[b] <= pos).astype(jnp.int32)
        return seg

    for j in range(NCHUNK):
        p = j % 2
        copies[p].wait()
        buf = bufs[p]
        cstart = base + j * CH

        sfirst = seg_of(cstart)
        slast = seg_of(cstart + (CH - 1))

        def b_body(b, _, buf=buf, cstart=cstart):
            cu_lo = _lane_extract(cu_val, lane, b)
            cu_hi = _lane_extract(cu_val, lane, b + 1)
            lo = jnp.clip(cu_lo - cstart, 0, CH)
            hi = jnp.clip(cu_hi - cstart, 0, CH)
            for g in range(G):
                def r_body(r, carry, buf=buf, g=g):
                    return tuple(
                        carry[k] + buf[r, pl.ds(g * GW + k * L, L)]
                        for k in range(GS)
                    )
                carry = lax.fori_loop(
                    lo, hi, r_body,
                    tuple(jnp.zeros((L,), jnp.float32) for _ in range(GS)))
                for k in range(GS):
                    sl = pl.ds(b * D + g * GW + k * L, L)
                    acc[sl] = acc[sl] + carry[k]
            return 0

        lax.fori_loop(sfirst, slast + 1, b_body, 0)

    def w_body(b, _):
        pltpu.async_copy(acc.at[pl.ds(b * D, D)], pacc_out.at[wid, b], sem0)
        return 0
    lax.fori_loop(0, NB, w_body, 0)

    def wdrain_body(b, _):
        pltpu.make_async_copy(acc.at[pl.ds(0, D)], pacc_out.at[0, 0],
                              sem0).wait()
        return 0
    lax.fori_loop(0, NB, wdrain_body, 0)

    @pl.when(yv_worker)
    def _():
        yv_copy[0].wait()
        pltpu.sync_copy(yvbuf, yvcas_out)


def _tc_sum_body(cu_ref, mes_ref, out_ref):
    j = pl.program_id(0)

    @pl.when(j == 0)
    def _():
        out_ref[...] = jnp.zeros_like(out_ref)

    rows = lax.broadcasted_iota(jnp.int32, (NB, TCR), 1) + j * TCR
    lo = jnp.stack([cu_ref[b] for b in range(NB)])[:, None]
    hi = jnp.stack([cu_ref[b + 1] for b in range(NB)])[:, None]
    m = jnp.logical_and(lo <= rows, rows < hi).astype(jnp.float32)
    out_ref[...] += lax.dot(m, mes_ref[...],
                            preferred_element_type=jnp.float32)


def _tc_combine_body(cu_ref, pacc_ref, ptc_ref, out_ref):
    cnt = jnp.stack([cu_ref[b + 1] - cu_ref[b] for b in range(NB)])
    cntf = cnt[:, None].astype(jnp.float32)
    total = jnp.sum(pacc_ref[...], axis=0) + ptc_ref[...]
    out_ref[...] = total / cntf


@jax.jit
def _run(mes_update, yv, cu_seqlens):
    mesh = plsc.VectorSubcoreMesh(core_axis_name="c", subcore_axis_name="s", num_cores=NC)

    params = pltpu.CompilerParams(needs_layout_passes=False)
    sc_kernel = pl.kernel(
        _sc_body,
        mesh=mesh,
        compiler_params=params,
        out_type=[
            jax.ShapeDtypeStruct((NW, NB, D), jnp.float32),
            jax.ShapeDtypeStruct((NB, D), jnp.float32),
        ],
        scratch_types=[
            pltpu.VMEM((CH, D), jnp.float32),
            pltpu.VMEM((CH, D), jnp.float32),
            pltpu.VMEM((NB * D,), jnp.float32),
            pltpu.VMEM((L,), jnp.int32),
            pltpu.VMEM((NB, D), jnp.float32),
            pltpu.SemaphoreType.DMA,
            pltpu.SemaphoreType.DMA,
            pltpu.SemaphoreType.DMA,
        ],
    )
    ptc = pl.pallas_call(
        _tc_sum_body,
        grid=(TC_ROWS // TCR,),
        in_specs=[
            pl.BlockSpec(memory_space=pltpu.SMEM),
            pl.BlockSpec((TCR, D), lambda j: (j, 0)),
        ],
        out_specs=pl.BlockSpec((NB, D), lambda j: (0, 0)),
        out_shape=jax.ShapeDtypeStruct((NB, D), jnp.float32),
    )(cu_seqlens, mes_update)

    pacc, yv_cas = sc_kernel(mes_update, yv, cu_seqlens)

    cas_mean = pl.pallas_call(
        _tc_combine_body,
        in_specs=[
            pl.BlockSpec(memory_space=pltpu.SMEM),
            pl.BlockSpec((NW, NB, D)),
            pl.BlockSpec((NB, D)),
        ],
        out_specs=pl.BlockSpec((NB, D)),
        out_shape=jax.ShapeDtypeStruct((NB, D), jnp.float32),
    )(cu_seqlens, pacc, ptc)

    return cas_mean, yv_cas


def kernel(mes_update, yv, cu_seqlens):
    return _run(mes_update, yv, cu_seqlens.astype(jnp.int32))


# speedup vs baseline: 1.1499x; 1.0726x over previous
import jax
import jax.numpy as jnp
from jax import lax
from jax.experimental import pallas as pl
from jax.experimental.pallas import tpu as pltpu
from jax.experimental.pallas import tpu_sc as plsc

TOTAL = 8192
D = 1024
NB = 8
NC = 1
NS = 16
NW = NC * NS
TC_ROWS = 7168
TCR = 1024
SC_ROWS = TOTAL - TC_ROWS
RPW = SC_ROWS // NW
CH = RPW
NCHUNK = RPW // CH
L = 16
G = 2
GW = D // G
GS = GW // L
STRIP = NB * D // NS


def _lane_extract(vec, lane, i):
    return jnp.sum(jnp.where(lane == i, vec, 0))


def _sc_body(mes_hbm, yv_hbm, cu_hbm,
             pacc_out, yvcas_out,
             buf0, buf1, acc, cuv, yvbuf,
             sem0, sem1, semyv):
    c = lax.axis_index("c")
    s = lax.axis_index("s")
    wid = s * NC + c
    base = TC_ROWS + wid * RPW

    bufs = (buf0, buf1)
    sems = (sem0, sem1)
    copies = [None, None]
    copies[0] = pltpu.async_copy(
        mes_hbm.at[pl.ds(base, CH)], buf0, sem0)
    if NCHUNK > 1:
        copies[1] = pltpu.async_copy(
            mes_hbm.at[pl.ds(base + CH, CH)], buf1, sem1)

    pltpu.sync_copy(cu_hbm, cuv.at[pl.ds(0, NB + 1)])

    yv_worker = jnp.logical_and(c == 0, s == 1)
    yv_copy = [None]

    @pl.when(yv_worker)
    def _():
        yv_copy[0] = pltpu.async_copy(
            yv_hbm.at[cuv.at[pl.ds(0, NB)]], yvbuf, semyv)

    def z_body(i, _):
        for k in range(8):
            acc[pl.ds(i * 128 + k * L, L)] = jnp.zeros((L,), jnp.float32)
        return 0
    lax.fori_loop(0, NB * D // 128, z_body, 0)

    cu_val = cuv[...]
    lane = lax.iota(jnp.int32, L)
    cub = [_lane_extract(cu_val, lane, b) for b in range(1, NB + 1)]

    def seg_of(pos):
        seg = jnp.int32(0)
        for b in range(NB - 1):
            seg = seg + (cub[b] <= pos).astype(jnp.int32)
        return seg

    for j in range(NCHUNK):
        p = j % 2
        copies[p].wait()
        buf = bufs[p]
        cstart = base + j * CH

        sfirst = seg_of(cstart)
        slast = seg_of(cstart + (CH - 1))

        def b_body(b, _, buf=buf, cstart=cstart):
            cu_lo = _lane_extract(cu_val, lane, b)
            cu_hi = _lane_extract(cu_val, lane, b + 1)
            lo = jnp.clip(cu_lo - cstart, 0, CH)
            hi = jnp.clip(cu_hi - cstart, 0, CH)
            for g in range(G):
                def r_body(r, carry, buf=buf, g=g):
                    return tuple(
                        carry[k] + buf[r, pl.ds(g * GW + k * L, L)]
                        for k in range(GS)
                    )
                carry = lax.fori_loop(
                    lo, hi, r_body,
                    tuple(jnp.zeros((L,), jnp.float32) for _ in range(GS)))
                for k in range(GS):
                    sl = pl.ds(b * D + g * GW + k * L, L)
                    acc[sl] = acc[sl] + carry[k]
            return 0

        lax.fori_loop(sfirst, slast + 1, b_body, 0)

    def w_body(b, _):
        pltpu.async_copy(acc.at[pl.ds(b * D, D)], pacc_out.at[wid, b], sem0)
        return 0
    lax.fori_loop(0, NB, w_body, 0)

    def wdrain_body(b, _):
        pltpu.make_async_copy(acc.at[pl.ds(0, D)], pacc_out.at[0, 0],
                              sem0).wait()
        return 0
    lax.fori_loop(0, NB, wdrain_body, 0)

    @pl.when(yv_worker)
    def _():
        yv_copy[0].wait()
        pltpu.sync_copy(yvbuf, yvcas_out)


def _tc_sum_body(cu_ref, mes_ref, out_ref):
    j = pl.program_id(0)

    @pl.when(j == 0)
    def _():
        out_ref[...] = jnp.zeros_like(out_ref)

    rows = lax.broadcasted_iota(jnp.int32, (NB, TCR), 1) + j * TCR
    lo = jnp.stack([cu_ref[b] for b in range(NB)])[:, None]
    hi = jnp.stack([cu_ref[b + 1] for b in range(NB)])[:, None]
    m = jnp.logical_and(lo <= rows, rows < hi).astype(jnp.float32)
    out_ref[...] += lax.dot(m, mes_ref[...],
                            preferred_element_type=jnp.float32)


def _tc_combine_body(cu_ref, pacc_ref, ptc_ref, out_ref):
    cnt = jnp.stack([cu_ref[b + 1] - cu_ref[b] for b in range(NB)])
    cntf = cnt[:, None].astype(jnp.float32)
    total = jnp.sum(pacc_ref[...], axis=0) + ptc_ref[...]
    out_ref[...] = total / cntf


@jax.jit
def _run(mes_update, yv, cu_seqlens):
    mesh = plsc.VectorSubcoreMesh(core_axis_name="c", subcore_axis_name="s", num_cores=NC)

    params = pltpu.CompilerParams(needs_layout_passes=False)
    sc_kernel = pl.kernel(
        _sc_body,
        mesh=mesh,
        compiler_params=params,
        out_type=[
            jax.ShapeDtypeStruct((NW, NB, D), jnp.float32),
            jax.ShapeDtypeStruct((NB, D), jnp.float32),
        ],
        scratch_types=[
            pltpu.VMEM((CH, D), jnp.float32),
            pltpu.VMEM((CH, D), jnp.float32),
            pltpu.VMEM((NB * D,), jnp.float32),
            pltpu.VMEM((L,), jnp.int32),
            pltpu.VMEM((NB, D), jnp.float32),
            pltpu.SemaphoreType.DMA,
            pltpu.SemaphoreType.DMA,
            pltpu.SemaphoreType.DMA,
        ],
    )
    ptc = pl.pallas_call(
        _tc_sum_body,
        grid=(TC_ROWS // TCR,),
        in_specs=[
            pl.BlockSpec(memory_space=pltpu.SMEM),
            pl.BlockSpec((TCR, D), lambda j: (j, 0)),
        ],
        out_specs=pl.BlockSpec((NB, D), lambda j: (0, 0)),
        out_shape=jax.ShapeDtypeStruct((NB, D), jnp.float32),
    )(cu_seqlens, mes_update)

    pacc, yv_cas = sc_kernel(mes_update, yv, cu_seqlens)

    cas_mean = pl.pallas_call(
        _tc_combine_body,
        in_specs=[
            pl.BlockSpec(memory_space=pltpu.SMEM),
            pl.BlockSpec((NW, NB, D)),
            pl.BlockSpec((NB, D)),
        ],
        out_specs=pl.BlockSpec((NB, D)),
        out_shape=jax.ShapeDtypeStruct((NB, D), jnp.float32),
    )(cu_seqlens, pacc, ptc)

    return cas_mean, yv_cas


def kernel(mes_update, yv, cu_seqlens):
    return _run(mes_update, yv, cu_seqlens.astype(jnp.int32))
